# baseline (speedup 1.0000x reference)
import functools

import jax
import jax.numpy as jnp
from jax import lax
from jax.experimental import pallas as pl
from jax.experimental.pallas import tpu as pltpu, tpu_sc as plsc

N_NODE = 10000
NP = 10112
D = 128
OUT = 64

E_BIP = 320000
E_SELF = 10000
CH_M = 2560
CH_S = 128
PAD_M = CH_M * 128 - E_BIP
PAD_S = CH_S * 128 - E_SELF
OFF_M, OFF_R, OFF_C, OFF_G = 0, CH_M, 2 * CH_M, 2 * CH_M + CH_S
CH_TOT = 2 * CH_M + 2 * CH_S
KB_MAX = 40

_MM = functools.partial(jnp.dot, preferred_element_type=jnp.float32,
                        precision=lax.Precision.DEFAULT)


def _stage_a_body(xg_ref, xc_ref, wg_ref, wc_ref, b_ref,
                  ym_ref, ycs_ref, yrev_ref, ygs_ref, r_ref):
    og = _MM(xg_ref[...], wg_ref[...])
    oc = _MM(xc_ref[...], wc_ref[...])
    ym_ref[...] = og[:, :D]
    ygs_ref[...] = og[:, D:2 * D]
    r_ref[1] = og[:, 2 * D:] + b_ref[1]
    ycs_ref[...] = oc[:, :D]
    yrev_ref[...] = oc[:, D:2 * D]
    r_ref[0] = oc[:, 2 * D:] + b_ref[0]


def _stage_b_body(a_ref, wg_ref, wc_ref, b_ref, ym_ref, ycs_ref, r_ref):
    hc = jnp.maximum(a_ref[0], 0.0)
    hg = jnp.maximum(a_ref[1], 0.0)
    oc = _MM(hc, wc_ref[...])
    ym_ref[...] = _MM(hg, wg_ref[...])
    ycs_ref[...] = oc[:, :D]
    r_ref[0] = oc[:, D:] + b_ref[0]
    r_ref[1] = jnp.zeros((_ROWS_BLK, D), jnp.float32)


def _stage_c_body(a_ref, w_ref, b_ref, o_ref):
    h = jnp.maximum(a_ref[0] + a_ref[1], 0.0)
    o_ref[...] = _MM(h, w_ref[...]) + b_ref[0]


_ROWS_BLK = 1000
_GRID = N_NODE // _ROWS_BLK
_TBL_SPEC = pl.BlockSpec((_ROWS_BLK, D), lambda i: (i, 0))
_TBL_SHAPE = jax.ShapeDtypeStruct((N_NODE, D), jnp.float32)
_R_SPEC = pl.BlockSpec((2, _ROWS_BLK, D), lambda i: (0, i, 0))
_R_SHAPE = jax.ShapeDtypeStruct((2, NP, D), jnp.float32)


def _stage_a(xg, xc, wg, wc, b2):
    return pl.pallas_call(
        _stage_a_body,
        grid=(_GRID,),
        in_specs=[
            pl.BlockSpec((_ROWS_BLK, D), lambda i: (i, 0)),
            pl.BlockSpec((_ROWS_BLK, D), lambda i: (i, 0)),
            pl.BlockSpec((D, 3 * D), lambda i: (0, 0)),
            pl.BlockSpec((D, 3 * D), lambda i: (0, 0)),
            pl.BlockSpec((2, D), lambda i: (0, 0)),
        ],
        out_specs=[_TBL_SPEC, _TBL_SPEC, _TBL_SPEC, _TBL_SPEC, _R_SPEC],
        out_shape=[_TBL_SHAPE, _TBL_SHAPE, _TBL_SHAPE, _TBL_SHAPE, _R_SHAPE],
    )(xg, xc, wg, wc, b2)


def _stage_b(acc, wg, wc, b1):
    return pl.pallas_call(
        _stage_b_body,
        grid=(_GRID,),
        in_specs=[
            pl.BlockSpec((2, _ROWS_BLK, D), lambda i: (0, i, 0)),
            pl.BlockSpec((D, D), lambda i: (0, 0)),
            pl.BlockSpec((D, 2 * D), lambda i: (0, 0)),
            pl.BlockSpec((1, D), lambda i: (0, 0)),
        ],
        out_specs=[_TBL_SPEC, _TBL_SPEC, _R_SPEC],
        out_shape=[_TBL_SHAPE, _TBL_SHAPE, _R_SHAPE],
    )(acc, wg, wc, b1)


def _stage_c(acc, w, b1):
    return pl.pallas_call(
        _stage_c_body,
        grid=(_GRID,),
        in_specs=[
            pl.BlockSpec((2, _ROWS_BLK, D), lambda i: (0, i, 0)),
            pl.BlockSpec((D, D), lambda i: (0, 0)),
            pl.BlockSpec((1, D), lambda i: (0, 0)),
        ],
        out_specs=pl.BlockSpec((_ROWS_BLK, D), lambda i: (i, 0)),
        out_shape=jax.ShapeDtypeStruct((N_NODE, D), jnp.float32),
    )(acc, w, b1)


def _run_rel(src2d, dst2d, tbl, src_v, dst_v, rows_a, rows_b,
             acc_sh, sem_a, sem_b, sem_sa, sem_sb, start, kb, nb):
    for b in range(nb):
        base = start + b * kb
        pltpu.sync_copy(src2d.at[pl.ds(base, kb)], src_v.at[pl.ds(0, kb)])
        pltpu.sync_copy(dst2d.at[pl.ds(base, kb)], dst_v.at[pl.ds(0, kb)])

        pltpu.async_copy(tbl.at[src_v.at[0]], rows_a, sem_a)
        pltpu.async_copy(tbl.at[src_v.at[1]], rows_b, sem_b)

        def body(i, carry):
            j = i * 2
            pltpu.make_async_copy(tbl.at[src_v.at[j]], rows_a, sem_a).wait()
            pltpu.async_copy(rows_a, acc_sh.at[dst_v.at[j]], sem_sa,
                             add=True)
            pltpu.make_async_copy(tbl.at[src_v.at[j + 1]], rows_b,
                                  sem_b).wait()
            pltpu.async_copy(rows_b, acc_sh.at[dst_v.at[j + 1]], sem_sb,
                             add=True)
            pltpu.make_async_copy(rows_a, acc_sh.at[dst_v.at[j]],
                                  sem_sa).wait()
            pltpu.async_copy(tbl.at[src_v.at[j + 2]], rows_a, sem_a)
            pltpu.make_async_copy(rows_b, acc_sh.at[dst_v.at[j + 1]],
                                  sem_sb).wait()
            pltpu.async_copy(tbl.at[src_v.at[j + 3]], rows_b, sem_b)
            return carry

        lax.fori_loop(0, (kb - 2) // 2, body, 0)

        pltpu.make_async_copy(tbl.at[src_v.at[kb - 2]], rows_a, sem_a).wait()
        pltpu.sync_copy(rows_a, acc_sh.at[dst_v.at[kb - 2]], add=True)
        pltpu.make_async_copy(tbl.at[src_v.at[kb - 1]], rows_b, sem_b).wait()
        pltpu.sync_copy(rows_b, acc_sh.at[dst_v.at[kb - 1]], add=True)


_MESH = plsc.VectorSubcoreMesh(core_axis_name="c", subcore_axis_name="s")
_ROWS_PER_TILE = NP // 16
_SC_SCRATCH = [
    pltpu.VMEM((KB_MAX, 128), jnp.int32),
    pltpu.VMEM((KB_MAX, 128), jnp.int32),
    pltpu.VMEM((128, D), jnp.float32),
    pltpu.VMEM((128, D), jnp.float32),
    pltpu.VMEM_SHARED((NP, D), jnp.float32),
    pltpu.SemaphoreType.DMA,
    pltpu.SemaphoreType.DMA,
    pltpu.SemaphoreType.DMA,
    pltpu.SemaphoreType.DMA,
]


@functools.partial(
    pl.kernel, mesh=_MESH,
    out_type=jax.ShapeDtypeStruct((2, NP, D), jnp.float32),
    scratch_types=_SC_SCRATCH,
)
def _sc_layer1(src2d, dst2d, tm, tcs, trev, tgs, init_hbm, out_hbm,
               src_v, dst_v, rows_a, rows_b, acc_sh, sem_a, sem_b, sem_sa, sem_sb):
    c = lax.axis_index("c")
    s = lax.axis_index("s")
    row0 = s * _ROWS_PER_TILE
    pltpu.sync_copy(init_hbm.at[c, pl.ds(row0, _ROWS_PER_TILE)],
                    acc_sh.at[pl.ds(row0, _ROWS_PER_TILE)])
    plsc.subcore_barrier()

    args = (src_v, dst_v, rows_a, rows_b, acc_sh, sem_a, sem_b, sem_sa, sem_sb)

    @pl.when(c == 0)
    def _cell_side():
        _run_rel(src2d, dst2d, tm, *args, OFF_M + s * (CH_M // 16), 40, 4)
        _run_rel(src2d, dst2d, tcs, *args, OFF_C + s * (CH_S // 16), 8, 1)

    @pl.when(c == 1)
    def _gene_side():
        _run_rel(src2d, dst2d, trev, *args, OFF_R + s * (CH_M // 16), 40, 4)
        _run_rel(src2d, dst2d, tgs, *args, OFF_G + s * (CH_S // 16), 8, 1)

    plsc.subcore_barrier()
    pltpu.sync_copy(acc_sh.at[pl.ds(row0, _ROWS_PER_TILE)],
                    out_hbm.at[c, pl.ds(row0, _ROWS_PER_TILE)])


@functools.partial(
    pl.kernel, mesh=_MESH,
    out_type=jax.ShapeDtypeStruct((2, NP, D), jnp.float32),
    scratch_types=_SC_SCRATCH,
)
def _sc_layer2(src2d, dst2d, tm, tcs, init_hbm, out_hbm,
               src_v, dst_v, rows_a, rows_b, acc_sh, sem_a, sem_b, sem_sa, sem_sb):
    c = lax.axis_index("c")
    s = lax.axis_index("s")
    row0 = s * _ROWS_PER_TILE
    pltpu.sync_copy(init_hbm.at[c, pl.ds(row0, _ROWS_PER_TILE)],
                    acc_sh.at[pl.ds(row0, _ROWS_PER_TILE)])
    plsc.subcore_barrier()

    args = (src_v, dst_v, rows_a, rows_b, acc_sh, sem_a, sem_b, sem_sa, sem_sb)
    _run_rel(src2d, dst2d, tm, *args,
             OFF_M + c * (CH_M // 2) + s * (CH_M // 32), 40, 2)

    @pl.when(s < 8)
    def _cs():
        _run_rel(src2d, dst2d, tcs, *args, OFF_C + c * (CH_S // 2) + s * 8,
                 8, 1)

    plsc.subcore_barrier()
    pltpu.sync_copy(acc_sh.at[pl.ds(row0, _ROWS_PER_TILE)],
                    out_hbm.at[c, pl.ds(row0, _ROWS_PER_TILE)])


def kernel(x_gene, x_cell_type, edge_index_marker, edge_index_rev,
           edge_index_gene_self, edge_index_cell_self, params, lin):
    arm = jnp.arange(PAD_M, dtype=jnp.int32)
    ars = jnp.arange(PAD_S, dtype=jnp.int32)
    psrc_m, psrc_s = arm & 8191, ars & 8191
    pdst_m, pdst_s = N_NODE + (arm & 63), N_NODE + (ars & 63)
    srcs = jnp.concatenate([
        edge_index_marker[0], psrc_m, edge_index_rev[0], psrc_m,
        edge_index_cell_self[0], psrc_s, edge_index_gene_self[0], psrc_s,
    ]).reshape(CH_TOT, 128)
    dsts = jnp.concatenate([
        edge_index_marker[1], pdst_m, edge_index_rev[1], pdst_m,
        edge_index_cell_self[1], pdst_s, edge_index_gene_self[1], pdst_s,
    ]).reshape(CH_TOT, 128)

    p0, p1 = params
    wg0 = jnp.concatenate([
        p0["marker"]["W_rel"].T, p0["gene_self"]["W_rel"].T,
        (p0["rev"]["W_root"] + p0["gene_self"]["W_root"]).T,
    ], axis=1)
    wc0 = jnp.concatenate([
        p0["cell_self"]["W_rel"].T, p0["rev"]["W_rel"].T,
        (p0["marker"]["W_root"] + p0["cell_self"]["W_root"]).T,
    ], axis=1)
    b2 = jnp.stack([
        p0["marker"]["b_rel"] + p0["cell_self"]["b_rel"],
        p0["rev"]["b_rel"] + p0["gene_self"]["b_rel"],
    ])
    wg1 = p1["marker"]["W_rel"].T
    wc1 = jnp.concatenate([
        p1["cell_self"]["W_rel"].T,
        (p1["marker"]["W_root"] + p1["cell_self"]["W_root"]).T,
    ], axis=1)
    b1 = (p1["marker"]["b_rel"] + p1["cell_self"]["b_rel"]).reshape(1, D)
    w_lin = jnp.zeros((D, D), jnp.float32).at[:, :OUT].set(lin["W"].T)
    b_lin = jnp.zeros((1, D), jnp.float32).at[0, :OUT].set(lin["b"])

    ym0, ycs0, yrev0, ygs0, r0 = _stage_a(x_gene, x_cell_type, wg0, wc0, b2)
    acc1 = _sc_layer1(srcs, dsts, ym0, ycs0, yrev0, ygs0, r0)

    ym1, ycs1, r1 = _stage_b(acc1, wg1, wc1, b1)
    acc2 = _sc_layer2(srcs, dsts, ym1, ycs1, r1)

    out = _stage_c(acc2, w_lin, b_lin)
    return out[:, :OUT]


# speedup vs baseline: 1.2460x; 1.2460x over previous
import functools

import jax
import jax.numpy as jnp
from jax import lax
from jax.experimental import pallas as pl
from jax.experimental.pallas import tpu as pltpu, tpu_sc as plsc

N_NODE = 10000
NP = 10112
D = 128
OUT = 64

E_BIP = 320000
E_SELF = 10000
CH_M = 2560
CH_S = 128
PAD_M = CH_M * 128 - E_BIP
PAD_S = CH_S * 128 - E_SELF
OFF_M, OFF_R, OFF_C, OFF_G = 0, CH_M, 2 * CH_M, 2 * CH_M + CH_S
CH_TOT = 2 * CH_M + 2 * CH_S
KB_MAX = 40

_MM = functools.partial(jnp.dot, preferred_element_type=jnp.float32,
                        precision=lax.Precision.DEFAULT)


def _stage_a_body(xg_ref, xc_ref, wg_ref, wc_ref, b_ref,
                  ym_ref, ycs_ref, yrev_ref, ygs_ref, r_ref):
    og = _MM(xg_ref[...], wg_ref[...])
    oc = _MM(xc_ref[...], wc_ref[...])
    ym_ref[...] = og[:, :D]
    ygs_ref[...] = og[:, D:2 * D]
    r_ref[1] = og[:, 2 * D:] + b_ref[1]
    ycs_ref[...] = oc[:, :D]
    yrev_ref[...] = oc[:, D:2 * D]
    r_ref[0] = oc[:, 2 * D:] + b_ref[0]


def _stage_b_body(a_ref, wg_ref, wc_ref, b_ref, ym_ref, ycs_ref, r_ref):
    hc = jnp.maximum(a_ref[0], 0.0)
    hg = jnp.maximum(a_ref[1], 0.0)
    oc = _MM(hc, wc_ref[...])
    ym_ref[...] = _MM(hg, wg_ref[...])
    ycs_ref[...] = oc[:, :D]
    r_ref[0] = oc[:, D:] + b_ref[0]
    r_ref[1] = jnp.zeros((_ROWS_BLK, D), jnp.float32)


def _stage_c_body(a_ref, w_ref, b_ref, o_ref):
    h = jnp.maximum(a_ref[0] + a_ref[1], 0.0)
    o_ref[...] = _MM(h, w_ref[...]) + b_ref[0]


_ROWS_BLK = 1000
_GRID = N_NODE // _ROWS_BLK
_TBL_SPEC = pl.BlockSpec((_ROWS_BLK, D), lambda i: (i, 0))
_TBL_SHAPE = jax.ShapeDtypeStruct((N_NODE, D), jnp.float32)
_R_SPEC = pl.BlockSpec((2, _ROWS_BLK, D), lambda i: (0, i, 0))
_R_SHAPE = jax.ShapeDtypeStruct((2, NP, D), jnp.float32)


def _stage_a(xg, xc, wg, wc, b2):
    return pl.pallas_call(
        _stage_a_body,
        grid=(_GRID,),
        in_specs=[
            pl.BlockSpec((_ROWS_BLK, D), lambda i: (i, 0)),
            pl.BlockSpec((_ROWS_BLK, D), lambda i: (i, 0)),
            pl.BlockSpec((D, 3 * D), lambda i: (0, 0)),
            pl.BlockSpec((D, 3 * D), lambda i: (0, 0)),
            pl.BlockSpec((2, D), lambda i: (0, 0)),
        ],
        out_specs=[_TBL_SPEC, _TBL_SPEC, _TBL_SPEC, _TBL_SPEC, _R_SPEC],
        out_shape=[_TBL_SHAPE, _TBL_SHAPE, _TBL_SHAPE, _TBL_SHAPE, _R_SHAPE],
    )(xg, xc, wg, wc, b2)


def _stage_b(acc, wg, wc, b1):
    return pl.pallas_call(
        _stage_b_body,
        grid=(_GRID,),
        in_specs=[
            pl.BlockSpec((2, _ROWS_BLK, D), lambda i: (0, i, 0)),
            pl.BlockSpec((D, D), lambda i: (0, 0)),
            pl.BlockSpec((D, 2 * D), lambda i: (0, 0)),
            pl.BlockSpec((1, D), lambda i: (0, 0)),
        ],
        out_specs=[_TBL_SPEC, _TBL_SPEC, _R_SPEC],
        out_shape=[_TBL_SHAPE, _TBL_SHAPE, _R_SHAPE],
    )(acc, wg, wc, b1)


def _stage_c(acc, w, b1):
    return pl.pallas_call(
        _stage_c_body,
        grid=(_GRID,),
        in_specs=[
            pl.BlockSpec((2, _ROWS_BLK, D), lambda i: (0, i, 0)),
            pl.BlockSpec((D, D), lambda i: (0, 0)),
            pl.BlockSpec((1, D), lambda i: (0, 0)),
        ],
        out_specs=pl.BlockSpec((_ROWS_BLK, D), lambda i: (i, 0)),
        out_shape=jax.ShapeDtypeStruct((N_NODE, D), jnp.float32),
    )(acc, w, b1)


def _run_rel(src2d, dst2d, tbl, src_v, dst_v, rows_a, rows_b,
             acc_sh, sem_a, sem_b, sem_sa, sem_sb, start, kb, nb):
    for b in range(nb):
        base = start + b * kb
        pltpu.sync_copy(src2d.at[pl.ds(base, kb)], src_v.at[pl.ds(0, kb)])
        pltpu.sync_copy(dst2d.at[pl.ds(base, kb)], dst_v.at[pl.ds(0, kb)])

        pltpu.async_copy(tbl.at[src_v.at[0]], rows_a, sem_a)
        pltpu.async_copy(tbl.at[src_v.at[1]], rows_b, sem_b)

        def body(i, carry):
            j = i * 2
            pltpu.make_async_copy(tbl.at[src_v.at[j]], rows_a, sem_a).wait()
            pltpu.sync_copy(rows_a, acc_sh.at[dst_v.at[j]], add=True)
            pltpu.async_copy(tbl.at[src_v.at[j + 2]], rows_a, sem_a)
            pltpu.make_async_copy(tbl.at[src_v.at[j + 1]], rows_b,
                                  sem_b).wait()
            pltpu.sync_copy(rows_b, acc_sh.at[dst_v.at[j + 1]], add=True)
            pltpu.async_copy(tbl.at[src_v.at[j + 3]], rows_b, sem_b)
            return carry

        lax.fori_loop(0, (kb - 2) // 2, body, 0)

        pltpu.make_async_copy(tbl.at[src_v.at[kb - 2]], rows_a, sem_a).wait()
        pltpu.sync_copy(rows_a, acc_sh.at[dst_v.at[kb - 2]], add=True)
        pltpu.make_async_copy(tbl.at[src_v.at[kb - 1]], rows_b, sem_b).wait()
        pltpu.sync_copy(rows_b, acc_sh.at[dst_v.at[kb - 1]], add=True)


_MESH = plsc.VectorSubcoreMesh(core_axis_name="c", subcore_axis_name="s")
_ROWS_PER_TILE = NP // 16
_SC_SCRATCH = [
    pltpu.VMEM((KB_MAX, 128), jnp.int32),
    pltpu.VMEM((KB_MAX, 128), jnp.int32),
    pltpu.VMEM((128, D), jnp.float32),
    pltpu.VMEM((128, D), jnp.float32),
    pltpu.VMEM_SHARED((NP, D), jnp.float32),
    pltpu.SemaphoreType.DMA,
    pltpu.SemaphoreType.DMA,
    pltpu.SemaphoreType.DMA,
    pltpu.SemaphoreType.DMA,
]


@functools.partial(
    pl.kernel, mesh=_MESH,
    out_type=jax.ShapeDtypeStruct((2, NP, D), jnp.float32),
    scratch_types=_SC_SCRATCH,
)
def _sc_layer1(src2d, dst2d, tm, tcs, trev, tgs, init_hbm, out_hbm,
               src_v, dst_v, rows_a, rows_b, acc_sh, sem_a, sem_b, sem_sa, sem_sb):
    c = lax.axis_index("c")
    s = lax.axis_index("s")
    row0 = s * _ROWS_PER_TILE
    pltpu.sync_copy(init_hbm.at[c, pl.ds(row0, _ROWS_PER_TILE)],
                    acc_sh.at[pl.ds(row0, _ROWS_PER_TILE)])
    plsc.subcore_barrier()

    args = (src_v, dst_v, rows_a, rows_b, acc_sh, sem_a, sem_b, sem_sa, sem_sb)

    @pl.when(c == 0)
    def _cell_side():
        _run_rel(src2d, dst2d, tm, *args, OFF_M + s * (CH_M // 16), 40, 4)
        _run_rel(src2d, dst2d, tcs, *args, OFF_C + s * (CH_S // 16), 8, 1)

    @pl.when(c == 1)
    def _gene_side():
        _run_rel(src2d, dst2d, trev, *args, OFF_R + s * (CH_M // 16), 40, 4)
        _run_rel(src2d, dst2d, tgs, *args, OFF_G + s * (CH_S // 16), 8, 1)

    plsc.subcore_barrier()
    pltpu.sync_copy(acc_sh.at[pl.ds(row0, _ROWS_PER_TILE)],
                    out_hbm.at[c, pl.ds(row0, _ROWS_PER_TILE)])


@functools.partial(
    pl.kernel, mesh=_MESH,
    out_type=jax.ShapeDtypeStruct((2, NP, D), jnp.float32),
    scratch_types=_SC_SCRATCH,
)
def _sc_layer2(src2d, dst2d, tm, tcs, init_hbm, out_hbm,
               src_v, dst_v, rows_a, rows_b, acc_sh, sem_a, sem_b, sem_sa, sem_sb):
    c = lax.axis_index("c")
    s = lax.axis_index("s")
    row0 = s * _ROWS_PER_TILE
    pltpu.sync_copy(init_hbm.at[c, pl.ds(row0, _ROWS_PER_TILE)],
                    acc_sh.at[pl.ds(row0, _ROWS_PER_TILE)])
    plsc.subcore_barrier()

    args = (src_v, dst_v, rows_a, rows_b, acc_sh, sem_a, sem_b, sem_sa, sem_sb)
    _run_rel(src2d, dst2d, tm, *args,
             OFF_M + c * (CH_M // 2) + s * (CH_M // 32), 40, 2)

    @pl.when(s < 8)
    def _cs():
        _run_rel(src2d, dst2d, tcs, *args, OFF_C + c * (CH_S // 2) + s * 8,
                 8, 1)

    plsc.subcore_barrier()
    pltpu.sync_copy(acc_sh.at[pl.ds(row0, _ROWS_PER_TILE)],
                    out_hbm.at[c, pl.ds(row0, _ROWS_PER_TILE)])


def kernel(x_gene, x_cell_type, edge_index_marker, edge_index_rev,
           edge_index_gene_self, edge_index_cell_self, params, lin):
    arm = jnp.arange(PAD_M, dtype=jnp.int32)
    ars = jnp.arange(PAD_S, dtype=jnp.int32)
    psrc_m, psrc_s = arm & 8191, ars & 8191
    pdst_m, pdst_s = N_NODE + (arm & 63), N_NODE + (ars & 63)
    srcs = jnp.concatenate([
        edge_index_marker[0], psrc_m, edge_index_rev[0], psrc_m,
        edge_index_cell_self[0], psrc_s, edge_index_gene_self[0], psrc_s,
    ]).reshape(CH_TOT, 128)
    dsts = jnp.concatenate([
        edge_index_marker[1], pdst_m, edge_index_rev[1], pdst_m,
        edge_index_cell_self[1], pdst_s, edge_index_gene_self[1], pdst_s,
    ]).reshape(CH_TOT, 128)

    p0, p1 = params
    wg0 = jnp.concatenate([
        p0["marker"]["W_rel"].T, p0["gene_self"]["W_rel"].T,
        (p0["rev"]["W_root"] + p0["gene_self"]["W_root"]).T,
    ], axis=1)
    wc0 = jnp.concatenate([
        p0["cell_self"]["W_rel"].T, p0["rev"]["W_rel"].T,
        (p0["marker"]["W_root"] + p0["cell_self"]["W_root"]).T,
    ], axis=1)
    b2 = jnp.stack([
        p0["marker"]["b_rel"] + p0["cell_self"]["b_rel"],
        p0["rev"]["b_rel"] + p0["gene_self"]["b_rel"],
    ])
    wg1 = p1["marker"]["W_rel"].T
    wc1 = jnp.concatenate([
        p1["cell_self"]["W_rel"].T,
        (p1["marker"]["W_root"] + p1["cell_self"]["W_root"]).T,
    ], axis=1)
    b1 = (p1["marker"]["b_rel"] + p1["cell_self"]["b_rel"]).reshape(1, D)
    w_lin = jnp.zeros((D, D), jnp.float32).at[:, :OUT].set(lin["W"].T)
    b_lin = jnp.zeros((1, D), jnp.float32).at[0, :OUT].set(lin["b"])

    ym0, ycs0, yrev0, ygs0, r0 = _stage_a(x_gene, x_cell_type, wg0, wc0, b2)
    acc1 = _sc_layer1(srcs, dsts, ym0, ycs0, yrev0, ygs0, r0)

    ym1, ycs1, r1 = _stage_b(acc1, wg1, wc1, b1)
    acc2 = _sc_layer2(srcs, dsts, ym1, ycs1, r1)

    out = _stage_c(acc2, w_lin, b_lin)
    return out[:, :OUT]
